# baseline (speedup 1.0000x reference)
import functools

import jax
import jax.numpy as jnp
from jax import lax
from jax.experimental import pallas as pl
from jax.experimental.pallas import tpu as pltpu
from jax.experimental.pallas import tpu_sc as plsc

N_NEIGHBORS = 15
TARGET_RATIO = 1.8
LAMBDA_REG = 0.1
N_SOURCES = 32
N_BF_ITERS = 20
INF = 1e10

_N = 4096
_K = 128
_BR = 256

_TOPK = 16
_NW = 32
_ROWS_PER_W = _N // _NW
_CHUNK = 8
_N_CHUNKS = _ROWS_PER_W // _CHUNK
_L = 16
_VPR = _N // _L


def _d2_kernel(x_blk, xt_full, sq_blk, sq_full, out):
    acc = jnp.dot(x_blk[...], xt_full[...], preferred_element_type=jnp.float32)
    out[...] = sq_blk[...].T + sq_full[...] - 2.0 * acc


def _pairwise_d2(x):
    sq = jnp.sum(x * x, axis=1)
    return pl.pallas_call(
        _d2_kernel,
        grid=(_N // _BR,),
        in_specs=[
            pl.BlockSpec((_BR, _K), lambda i: (i, 0)),
            pl.BlockSpec((_K, _N), lambda i: (0, 0)),
            pl.BlockSpec((1, _BR), lambda i: (0, i)),
            pl.BlockSpec((1, _N), lambda i: (0, 0)),
        ],
        out_specs=pl.BlockSpec((_BR, _N), lambda i: (i, 0)),
        out_shape=jax.ShapeDtypeStruct((_N, _N), jnp.float32),
    )(x, x.T, sq[None, :], sq[None, :])


def _topk_body(d2_hbm, val_hbm, idx_hbm, buf, oval, oidx, tref, tiref, t15ref):
    wid = lax.axis_index("s") * 2 + lax.axis_index("c")
    lane = lax.iota(jnp.int32, _L)
    last_lane = lane == _L - 1

    def chunk_body(c, _):
        row_base = wid * _ROWS_PER_W + c * _CHUNK
        pltpu.sync_copy(d2_hbm.at[pl.ds(row_base, _CHUNK)], buf)

        def row_body(r, _):
            def merge(j, carry):
                tval, tidx = carry
                v = buf[r, pl.ds(j * _L, _L)]
                i = j * _L + lane
                vs, is_ = plsc.sort_key_val(v, i)
                rv = lax.rev(tval, (0,))
                ri = lax.rev(tidx, (0,))
                sel = vs <= rv
                lo = jnp.minimum(vs, rv)
                li = jnp.where(sel, is_, ri)
                return tuple(plsc.sort_key_val(lo, li))

            t0 = (jnp.full((_L,), 1e30, jnp.float32),
                  jnp.zeros((_L,), jnp.int32))
            tval, tidx = lax.fori_loop(0, _L, merge, t0)
            tref[...] = tval
            tiref[...] = tidx
            t15ref[...] = jnp.broadcast_to(jnp.max(tval), (_L,))

            def scan(j, _):
                v = buf[r, pl.ds(j * _L, _L)]
                t15v = t15ref[...]

                @pl.when(jnp.any(v < t15v))
                def _():
                    def wcond(cw):
                        vv, t15 = cw
                        return jnp.any(vv < t15)

                    def wbody(cw):
                        vv, _ = cw
                        m = jnp.min(vv)
                        il = jnp.min(jnp.where(vv == m, lane, _L))
                        tv = jnp.where(last_lane, m, tref[...])
                        ti = jnp.where(last_lane, j * _L + il, tiref[...])
                        tv, ti = plsc.sort_key_val(tv, ti)
                        tref[...] = tv
                        tiref[...] = ti
                        nt15 = jnp.broadcast_to(jnp.max(tv), (_L,))
                        t15ref[...] = nt15
                        vv = jnp.where(lane == il, 1e30, vv)
                        return vv, nt15

                    lax.while_loop(wcond, wbody, (v, t15v))
                return 0

            lax.fori_loop(_L, _VPR, scan, 0)
            oval[r, :] = tref[...]
            oidx[r, :] = tiref[...]
            return 0

        lax.fori_loop(0, _CHUNK, row_body, 0)
        pltpu.sync_copy(oval, val_hbm.at[pl.ds(row_base, _CHUNK)])
        pltpu.sync_copy(oidx, idx_hbm.at[pl.ds(row_base, _CHUNK)])
        return 0

    lax.fori_loop(0, _N_CHUNKS, chunk_body, 0)


def _sc_topk(d2):
    mesh = plsc.VectorSubcoreMesh(core_axis_name="c", subcore_axis_name="s")
    f = pl.kernel(
        _topk_body,
        out_type=(
            jax.ShapeDtypeStruct((_N, _TOPK), jnp.float32),
            jax.ShapeDtypeStruct((_N, _TOPK), jnp.int32),
        ),
        mesh=mesh,
        scratch_types=[
            pltpu.VMEM((_CHUNK, _N), jnp.float32),
            pltpu.VMEM((_CHUNK, _TOPK), jnp.float32),
            pltpu.VMEM((_CHUNK, _TOPK), jnp.int32),
            pltpu.VMEM((_L,), jnp.float32),
            pltpu.VMEM((_L,), jnp.int32),
            pltpu.VMEM((_L,), jnp.float32),
        ],
        compiler_params=pltpu.CompilerParams(needs_layout_passes=False),
    )
    return f(d2)


_BIG = 1e30


def _bf_body(ip_hbm, wT_hbm, dist_hbm, idx_res, w_res, dist_old, dist_new):
    wid = lax.axis_index("s") * 2 + lax.axis_index("c")
    lane = lax.iota(jnp.int32, _L)

    pltpu.sync_copy(ip_hbm, idx_res)
    pltpu.sync_copy(wT_hbm, w_res)

    def init_j(j, _):
        dist_old[pl.ds(j * _L, _L)] = jnp.full((_L,), INF, jnp.float32)
        return 0
    lax.fori_loop(0, _VPR, init_j, 0)
    dist_old[pl.ds((wid // _L) * _L, _L)] = jnp.where(
        lane == wid % _L, 0.0, INF)

    def bf_cond(c):
        it, changed = c
        return (it < N_BF_ITERS) & changed

    def bf_body(c):
        it, _ = c

        def copy_j(j, _):
            ds = pl.ds(j * _L, _L)
            dist_new[ds] = dist_old[ds]
            return 0
        lax.fori_loop(0, _VPR, copy_j, 0)

        def j_body(j, _):
            dsA = pl.ds(j * 2 * _L, _L)
            dsB = pl.ds(j * 2 * _L + _L, _L)
            mnA = dist_new[dsA]
            mnB = dist_new[dsB]
            doA = dist_old[dsA]
            doB = dist_old[dsB]
            for k in range(N_NEIGHBORS):
                v32 = idx_res[k, pl.ds(j * _L, _L)]
                ia = v32 & 0xFFFF
                ib = lax.shift_right_logical(v32, 16)
                wA = w_res[k, dsA]
                wB = w_res[k, dsB]
                mnA = jnp.minimum(mnA, plsc.load_gather(dist_old, [ia]) + wA)
                mnB = jnp.minimum(mnB, plsc.load_gather(dist_old, [ib]) + wB)
                candA = doA + wA
                candB = doB + wB
                lostA = candA < plsc.load_gather(dist_new, [ia])
                lostB = candB < plsc.load_gather(dist_new, [ib])

                @pl.when(jnp.any(lostA | lostB))
                def _():
                    def wbody(cw):
                        la, lb = cw
                        plsc.store_scatter(dist_new, [ia], candA, mask=la)
                        plsc.store_scatter(dist_new, [ib], candB, mask=lb)
                        ra = plsc.load_gather(dist_new, [ia])
                        rb = plsc.load_gather(dist_new, [ib])
                        return candA < ra, candB < rb
                    lax.while_loop(lambda cw: jnp.any(cw[0] | cw[1]),
                                   wbody, (lostA, lostB))
            dist_new[dsA] = jnp.minimum(dist_new[dsA], mnA)
            dist_new[dsB] = jnp.minimum(dist_new[dsB], mnB)
            return 0
        lax.fori_loop(0, _VPR // 2, j_body, 0)

        def diff_j(j, acc):
            ds = pl.ds(j * _L, _L)
            a = dist_new[ds]
            acc = jnp.maximum(acc, jnp.where(a != dist_old[ds], 1, 0))
            dist_old[ds] = a
            return acc
        accv = lax.fori_loop(0, _VPR, diff_j, jnp.zeros((_L,), jnp.int32))
        return it + 1, jnp.max(accv) > 0

    lax.while_loop(bf_cond, bf_body, (0, True))
    pltpu.sync_copy(dist_old, dist_hbm.at[wid])


def _sc_bf(knn_i_packed, knn_wT):
    mesh = plsc.VectorSubcoreMesh(core_axis_name="c", subcore_axis_name="s")
    f = pl.kernel(
        _bf_body,
        out_type=jax.ShapeDtypeStruct((N_SOURCES, _N), jnp.float32),
        mesh=mesh,
        scratch_types=[
            pltpu.VMEM((N_NEIGHBORS, _N // 2), jnp.int32),
            pltpu.VMEM((N_NEIGHBORS, _N), jnp.float32),
            pltpu.VMEM((_N,), jnp.float32),
            pltpu.VMEM((_N,), jnp.float32),
        ],
        compiler_params=pltpu.CompilerParams(needs_layout_passes=False,
                                             use_tc_tiling_on_sc=False),
    )
    return f(knn_i_packed, knn_wT)


def _sqrt_kernel(d2_blk, out):
    out[...] = jnp.sqrt(jnp.maximum(d2_blk[...], 0.0))


def _knn_sqrt(knn_d2):
    return pl.pallas_call(
        _sqrt_kernel,
        out_shape=jax.ShapeDtypeStruct(knn_d2.shape, jnp.float32),
    )(knn_d2)


def _loss_kernel(dist_blk, euc2_blk, out):
    dist = dist_blk[...]
    euc = jnp.sqrt(jnp.maximum(euc2_blk[...], 0.0))
    mask = (dist < INF * 0.5) & (euc > 1e-8)
    ratios = dist / jnp.maximum(euc, 1e-8)
    sq_err = jnp.where(mask, (ratios - TARGET_RATIO) ** 2, 0.0)
    cnt = jnp.maximum(jnp.sum(mask.astype(jnp.int32)), 1)
    loss = jnp.sum(sq_err) / cnt.astype(jnp.float32) * LAMBDA_REG
    out[...] = loss[None, None]


def _loss(dist, d2):
    return pl.pallas_call(
        _loss_kernel,
        grid=(1,),
        in_specs=[
            pl.BlockSpec((N_SOURCES, _N), lambda i: (0, 0)),
            pl.BlockSpec((N_SOURCES, _N), lambda i: (0, 0)),
        ],
        out_specs=pl.BlockSpec((1, 1), lambda i: (0, 0)),
        out_shape=jax.ShapeDtypeStruct((1, 1), jnp.float32),
    )(dist, d2)


def kernel(embeddings):
    x = lax.stop_gradient(embeddings)
    D2 = _pairwise_d2(x)
    knn_d2, knn_idx = _sc_topk(D2)
    knn_d = _knn_sqrt(knn_d2[:, 1:])
    knn_i = knn_idx[:, 1:]

    iT3 = knn_i.T.astype(jnp.int32).reshape(N_NEIGHBORS, _N // 32, 2, _L)
    ipacked = (iT3[:, :, 0, :] | (iT3[:, :, 1, :] << 16)).reshape(
        N_NEIGHBORS, _N // 2)
    dist = _sc_bf(ipacked, knn_d.T)
    loss = _loss(dist, D2[:N_SOURCES, :])
    return loss[0, 0]


# speedup vs baseline: 2.5666x; 2.5666x over previous
import functools

import jax
import jax.numpy as jnp
from jax import lax
from jax.experimental import pallas as pl
from jax.experimental.pallas import tpu as pltpu
from jax.experimental.pallas import tpu_sc as plsc

N_NEIGHBORS = 15
TARGET_RATIO = 1.8
LAMBDA_REG = 0.1
N_SOURCES = 32
N_BF_ITERS = 20
INF = 1e10

_N = 4096
_K = 128
_BR = 256

_TOPK = 16
_NW = 32
_ROWS_PER_W = _N // _NW
_CHUNK = 8
_N_CHUNKS = _ROWS_PER_W // _CHUNK
_L = 16
_VPR = _N // _L


def _d2_kernel(x_blk, xt_full, sq_blk, sq_full, out):
    acc = jnp.dot(x_blk[...], xt_full[...], preferred_element_type=jnp.float32)
    out[...] = sq_blk[...].T + sq_full[...] - 2.0 * acc


def _pairwise_d2(x):
    sq = jnp.sum(x * x, axis=1)
    return pl.pallas_call(
        _d2_kernel,
        grid=(_N // _BR,),
        in_specs=[
            pl.BlockSpec((_BR, _K), lambda i: (i, 0)),
            pl.BlockSpec((_K, _N), lambda i: (0, 0)),
            pl.BlockSpec((1, _BR), lambda i: (0, i)),
            pl.BlockSpec((1, _N), lambda i: (0, 0)),
        ],
        out_specs=pl.BlockSpec((_BR, _N), lambda i: (i, 0)),
        out_shape=jax.ShapeDtypeStruct((_N, _N), jnp.float32),
    )(x, x.T, sq[None, :], sq[None, :])


def _topk_body(d2_hbm, val_hbm, idx_hbm, buf, oval, oidx, tref, tiref, t15ref):
    wid = lax.axis_index("s") * 2 + lax.axis_index("c")
    lane = lax.iota(jnp.int32, _L)
    last_lane = lane == _L - 1

    def chunk_body(c, _):
        row_base = wid * _ROWS_PER_W + c * _CHUNK
        pltpu.sync_copy(d2_hbm.at[pl.ds(row_base, _CHUNK)], buf)

        def row_body(r, _):
            def one_merge(tval, tidx, v, i):
                vs, is_ = plsc.sort_key_val(v, i)
                rv = lax.rev(tval, (0,))
                ri = lax.rev(tidx, (0,))
                sel = vs <= rv
                lo = jnp.minimum(vs, rv)
                li = jnp.where(sel, is_, ri)
                return tuple(plsc.sort_key_val(lo, li))

            def merge2(j, carry):
                tvA, tiA, tvB, tiB = carry
                vA = buf[r, pl.ds(j * 2 * _L, _L)]
                vB = buf[r, pl.ds(j * 2 * _L + _L, _L)]
                tvA, tiA = one_merge(tvA, tiA, vA, j * 2 * _L + lane)
                tvB, tiB = one_merge(tvB, tiB, vB, j * 2 * _L + _L + lane)
                return tvA, tiA, tvB, tiB

            big = jnp.full((_L,), 1e30, jnp.float32)
            zero = jnp.zeros((_L,), jnp.int32)
            tvA, tiA, tvB, tiB = lax.fori_loop(
                0, _VPR // 2, merge2, (big, zero, big, zero))
            rv = lax.rev(tvB, (0,))
            ri = lax.rev(tiB, (0,))
            sel = tvA <= rv
            lo = jnp.minimum(tvA, rv)
            li = jnp.where(sel, tiA, ri)
            tval, tidx = plsc.sort_key_val(lo, li)
            oval[r, :] = tval
            oidx[r, :] = tidx
            return 0

        lax.fori_loop(0, _CHUNK, row_body, 0)
        pltpu.sync_copy(oval, val_hbm.at[pl.ds(row_base, _CHUNK)])
        pltpu.sync_copy(oidx, idx_hbm.at[pl.ds(row_base, _CHUNK)])
        return 0

    lax.fori_loop(0, _N_CHUNKS, chunk_body, 0)


def _sc_topk(d2):
    mesh = plsc.VectorSubcoreMesh(core_axis_name="c", subcore_axis_name="s")
    f = pl.kernel(
        _topk_body,
        out_type=(
            jax.ShapeDtypeStruct((_N, _TOPK), jnp.float32),
            jax.ShapeDtypeStruct((_N, _TOPK), jnp.int32),
        ),
        mesh=mesh,
        scratch_types=[
            pltpu.VMEM((_CHUNK, _N), jnp.float32),
            pltpu.VMEM((_CHUNK, _TOPK), jnp.float32),
            pltpu.VMEM((_CHUNK, _TOPK), jnp.int32),
            pltpu.VMEM((_L,), jnp.float32),
            pltpu.VMEM((_L,), jnp.int32),
            pltpu.VMEM((_L,), jnp.float32),
        ],
        compiler_params=pltpu.CompilerParams(needs_layout_passes=False),
    )
    return f(d2)


_BIG = 1e30


def _bf_body(ip_hbm, wT_hbm, dist_hbm, idx_res, w_res, dist_old, dist_new):
    wid = lax.axis_index("s") * 2 + lax.axis_index("c")
    lane = lax.iota(jnp.int32, _L)

    pltpu.sync_copy(ip_hbm, idx_res)
    pltpu.sync_copy(wT_hbm, w_res)

    def init_j(j, _):
        dist_old[pl.ds(j * _L, _L)] = jnp.full((_L,), INF, jnp.float32)
        return 0
    lax.fori_loop(0, _VPR, init_j, 0)
    dist_old[pl.ds((wid // _L) * _L, _L)] = jnp.where(
        lane == wid % _L, 0.0, INF)

    def bf_cond(c):
        it, changed = c
        return (it < N_BF_ITERS) & changed

    def bf_body(c):
        it, _ = c

        def copy_j(j, _):
            ds = pl.ds(j * _L, _L)
            dist_new[ds] = dist_old[ds]
            return 0
        lax.fori_loop(0, _VPR, copy_j, 0)

        def j_body(j, _):
            dsA = pl.ds(j * 2 * _L, _L)
            dsB = pl.ds(j * 2 * _L + _L, _L)
            mnA = dist_new[dsA]
            mnB = dist_new[dsB]
            doA = dist_old[dsA]
            doB = dist_old[dsB]
            for k in range(N_NEIGHBORS):
                v32 = idx_res[k, pl.ds(j * _L, _L)]
                ia = v32 & 0xFFFF
                ib = lax.shift_right_logical(v32, 16)
                wA = w_res[k, dsA]
                wB = w_res[k, dsB]
                mnA = jnp.minimum(mnA, plsc.load_gather(dist_old, [ia]) + wA)
                mnB = jnp.minimum(mnB, plsc.load_gather(dist_old, [ib]) + wB)
                candA = doA + wA
                candB = doB + wB
                lostA = candA < plsc.load_gather(dist_new, [ia])
                lostB = candB < plsc.load_gather(dist_new, [ib])

                @pl.when(jnp.any(lostA | lostB))
                def _():
                    def wbody(cw):
                        la, lb = cw
                        plsc.store_scatter(dist_new, [ia], candA, mask=la)
                        plsc.store_scatter(dist_new, [ib], candB, mask=lb)
                        ra = plsc.load_gather(dist_new, [ia])
                        rb = plsc.load_gather(dist_new, [ib])
                        return candA < ra, candB < rb
                    lax.while_loop(lambda cw: jnp.any(cw[0] | cw[1]),
                                   wbody, (lostA, lostB))
            dist_new[dsA] = jnp.minimum(dist_new[dsA], mnA)
            dist_new[dsB] = jnp.minimum(dist_new[dsB], mnB)
            return 0
        lax.fori_loop(0, _VPR // 2, j_body, 0)

        def diff_j(j, acc):
            ds = pl.ds(j * _L, _L)
            a = dist_new[ds]
            acc = jnp.maximum(acc, jnp.where(a != dist_old[ds], 1, 0))
            dist_old[ds] = a
            return acc
        accv = lax.fori_loop(0, _VPR, diff_j, jnp.zeros((_L,), jnp.int32))
        return it + 1, jnp.max(accv) > 0

    lax.while_loop(bf_cond, bf_body, (0, True))
    pltpu.sync_copy(dist_old, dist_hbm.at[wid])


def _sc_bf(knn_i_packed, knn_wT):
    mesh = plsc.VectorSubcoreMesh(core_axis_name="c", subcore_axis_name="s")
    f = pl.kernel(
        _bf_body,
        out_type=jax.ShapeDtypeStruct((N_SOURCES, _N), jnp.float32),
        mesh=mesh,
        scratch_types=[
            pltpu.VMEM((N_NEIGHBORS, _N // 2), jnp.int32),
            pltpu.VMEM((N_NEIGHBORS, _N), jnp.float32),
            pltpu.VMEM((_N,), jnp.float32),
            pltpu.VMEM((_N,), jnp.float32),
        ],
        compiler_params=pltpu.CompilerParams(needs_layout_passes=False,
                                             use_tc_tiling_on_sc=False),
    )
    return f(knn_i_packed, knn_wT)


def _sqrt_kernel(d2_blk, out):
    out[...] = jnp.sqrt(jnp.maximum(d2_blk[...], 0.0))


def _knn_sqrt(knn_d2):
    return pl.pallas_call(
        _sqrt_kernel,
        out_shape=jax.ShapeDtypeStruct(knn_d2.shape, jnp.float32),
    )(knn_d2)


def _loss_kernel(dist_blk, euc2_blk, out):
    dist = dist_blk[...]
    euc = jnp.sqrt(jnp.maximum(euc2_blk[...], 0.0))
    mask = (dist < INF * 0.5) & (euc > 1e-8)
    ratios = dist / jnp.maximum(euc, 1e-8)
    sq_err = jnp.where(mask, (ratios - TARGET_RATIO) ** 2, 0.0)
    cnt = jnp.maximum(jnp.sum(mask.astype(jnp.int32)), 1)
    loss = jnp.sum(sq_err) / cnt.astype(jnp.float32) * LAMBDA_REG
    out[...] = loss[None, None]


def _loss(dist, d2):
    return pl.pallas_call(
        _loss_kernel,
        grid=(1,),
        in_specs=[
            pl.BlockSpec((N_SOURCES, _N), lambda i: (0, 0)),
            pl.BlockSpec((N_SOURCES, _N), lambda i: (0, 0)),
        ],
        out_specs=pl.BlockSpec((1, 1), lambda i: (0, 0)),
        out_shape=jax.ShapeDtypeStruct((1, 1), jnp.float32),
    )(dist, d2)


def kernel(embeddings):
    x = lax.stop_gradient(embeddings)
    D2 = _pairwise_d2(x)
    knn_d2, knn_idx = _sc_topk(D2)
    knn_d = _knn_sqrt(knn_d2[:, 1:])
    knn_i = knn_idx[:, 1:]

    iT3 = knn_i.T.astype(jnp.int32).reshape(N_NEIGHBORS, _N // 32, 2, _L)
    ipacked = (iT3[:, :, 0, :] | (iT3[:, :, 1, :] << 16)).reshape(
        N_NEIGHBORS, _N // 2)
    dist = _sc_bf(ipacked, knn_d.T)
    loss = _loss(dist, D2[:N_SOURCES, :])
    return loss[0, 0]


# speedup vs baseline: 2.9638x; 1.1547x over previous
import functools

import jax
import jax.numpy as jnp
from jax import lax
from jax.experimental import pallas as pl
from jax.experimental.pallas import tpu as pltpu
from jax.experimental.pallas import tpu_sc as plsc

N_NEIGHBORS = 15
TARGET_RATIO = 1.8
LAMBDA_REG = 0.1
N_SOURCES = 32
N_BF_ITERS = 20
INF = 1e10

_N = 4096
_K = 128
_BR = 256

_TOPK = 16
_NW = 32
_ROWS_PER_W = _N // _NW
_CHUNK = 8
_N_CHUNKS = _ROWS_PER_W // _CHUNK
_L = 16
_VPR = _N // _L


def _d2_kernel(x_blk, xt_full, sq_blk, sq_full, out):
    acc = jnp.dot(x_blk[...], xt_full[...], preferred_element_type=jnp.float32)
    out[...] = sq_blk[...].T + sq_full[...] - 2.0 * acc


def _pairwise_d2(x):
    sq = jnp.sum(x * x, axis=1)
    return pl.pallas_call(
        _d2_kernel,
        grid=(_N // _BR,),
        in_specs=[
            pl.BlockSpec((_BR, _K), lambda i: (i, 0)),
            pl.BlockSpec((_K, _N), lambda i: (0, 0)),
            pl.BlockSpec((1, _BR), lambda i: (0, i)),
            pl.BlockSpec((1, _N), lambda i: (0, 0)),
        ],
        out_specs=pl.BlockSpec((_BR, _N), lambda i: (i, 0)),
        out_shape=jax.ShapeDtypeStruct((_N, _N), jnp.float32),
    )(x, x.T, sq[None, :], sq[None, :])


def _topk_body(d2_hbm, val_hbm, idx_hbm, buf, oval, oidx, tref, tiref, t15ref):
    wid = lax.axis_index("s") * 2 + lax.axis_index("c")
    lane = lax.iota(jnp.int32, _L)
    last_lane = lane == _L - 1

    def chunk_body(c, _):
        row_base = wid * _ROWS_PER_W + c * _CHUNK
        pltpu.sync_copy(d2_hbm.at[pl.ds(row_base, _CHUNK)], buf)

        def row_body(r, _):
            def one_merge(tval, tidx, v, i):
                vs, is_ = plsc.sort_key_val(v, i)
                rv = lax.rev(tval, (0,))
                ri = lax.rev(tidx, (0,))
                sel = vs <= rv
                lo = jnp.minimum(vs, rv)
                li = jnp.where(sel, is_, ri)
                return tuple(plsc.sort_key_val(lo, li))

            nch = 4

            def merge4(j, carry):
                out = []
                for q in range(nch):
                    tv, ti = carry[2 * q], carry[2 * q + 1]
                    base = (j * nch + q) * _L
                    v = buf[r, pl.ds(base, _L)]
                    tv, ti = one_merge(tv, ti, v, base + lane)
                    out += [tv, ti]
                return tuple(out)

            big = jnp.full((_L,), 1e30, jnp.float32)
            zero = jnp.zeros((_L,), jnp.int32)
            carry = lax.fori_loop(0, _VPR // nch, merge4,
                                  (big, zero) * nch)
            while len(carry) > 2:
                nxt = []
                for q in range(0, len(carry), 4):
                    tvA, tiA, tvB, tiB = carry[q:q + 4]
                    rv = lax.rev(tvB, (0,))
                    ri = lax.rev(tiB, (0,))
                    sel = tvA <= rv
                    lo = jnp.minimum(tvA, rv)
                    li = jnp.where(sel, tiA, ri)
                    nxt += list(plsc.sort_key_val(lo, li))
                carry = nxt
            oval[r, :] = carry[0]
            oidx[r, :] = carry[1]
            return 0

        lax.fori_loop(0, _CHUNK, row_body, 0)
        pltpu.sync_copy(oval, val_hbm.at[pl.ds(row_base, _CHUNK)])
        pltpu.sync_copy(oidx, idx_hbm.at[pl.ds(row_base, _CHUNK)])
        return 0

    lax.fori_loop(0, _N_CHUNKS, chunk_body, 0)


def _sc_topk(d2):
    mesh = plsc.VectorSubcoreMesh(core_axis_name="c", subcore_axis_name="s")
    f = pl.kernel(
        _topk_body,
        out_type=(
            jax.ShapeDtypeStruct((_N, _TOPK), jnp.float32),
            jax.ShapeDtypeStruct((_N, _TOPK), jnp.int32),
        ),
        mesh=mesh,
        scratch_types=[
            pltpu.VMEM((_CHUNK, _N), jnp.float32),
            pltpu.VMEM((_CHUNK, _TOPK), jnp.float32),
            pltpu.VMEM((_CHUNK, _TOPK), jnp.int32),
            pltpu.VMEM((_L,), jnp.float32),
            pltpu.VMEM((_L,), jnp.int32),
            pltpu.VMEM((_L,), jnp.float32),
        ],
        compiler_params=pltpu.CompilerParams(needs_layout_passes=False),
    )
    return f(d2)


_BIG = 1e30


def _bf_body(ip_hbm, wT_hbm, dist_hbm, idx_res, w_res, dist_old, dist_new):
    wid = lax.axis_index("s") * 2 + lax.axis_index("c")
    lane = lax.iota(jnp.int32, _L)

    pltpu.sync_copy(ip_hbm, idx_res)
    pltpu.sync_copy(wT_hbm, w_res)

    def init_j(j, _):
        dist_old[pl.ds(j * _L, _L)] = jnp.full((_L,), INF, jnp.float32)
        return 0
    lax.fori_loop(0, _VPR, init_j, 0)
    dist_old[pl.ds((wid // _L) * _L, _L)] = jnp.where(
        lane == wid % _L, 0.0, INF)

    def bf_cond(c):
        it, changed = c
        return (it < N_BF_ITERS) & changed

    def bf_body(c):
        it, _ = c

        def copy_j(j, _):
            ds = pl.ds(j * _L, _L)
            dist_new[ds] = dist_old[ds]
            return 0
        lax.fori_loop(0, _VPR, copy_j, 0)

        def j_body(j, _):
            dsA = pl.ds(j * 2 * _L, _L)
            dsB = pl.ds(j * 2 * _L + _L, _L)
            mnA = dist_new[dsA]
            mnB = dist_new[dsB]
            doA = dist_old[dsA]
            doB = dist_old[dsB]
            for k in range(N_NEIGHBORS):
                v32 = idx_res[k, pl.ds(j * _L, _L)]
                ia = v32 & 0xFFFF
                ib = lax.shift_right_logical(v32, 16)
                wA = w_res[k, dsA]
                wB = w_res[k, dsB]
                mnA = jnp.minimum(mnA, plsc.load_gather(dist_old, [ia]) + wA)
                mnB = jnp.minimum(mnB, plsc.load_gather(dist_old, [ib]) + wB)
                candA = doA + wA
                candB = doB + wB
                lostA = candA < plsc.load_gather(dist_new, [ia])
                lostB = candB < plsc.load_gather(dist_new, [ib])

                @pl.when(jnp.any(lostA | lostB))
                def _():
                    def wbody(cw):
                        la, lb = cw
                        plsc.store_scatter(dist_new, [ia], candA, mask=la)
                        plsc.store_scatter(dist_new, [ib], candB, mask=lb)
                        ra = plsc.load_gather(dist_new, [ia])
                        rb = plsc.load_gather(dist_new, [ib])
                        return candA < ra, candB < rb
                    lax.while_loop(lambda cw: jnp.any(cw[0] | cw[1]),
                                   wbody, (lostA, lostB))
            dist_new[dsA] = jnp.minimum(dist_new[dsA], mnA)
            dist_new[dsB] = jnp.minimum(dist_new[dsB], mnB)
            return 0
        lax.fori_loop(0, _VPR // 2, j_body, 0)

        def diff_j(j, acc):
            ds = pl.ds(j * _L, _L)
            a = dist_new[ds]
            acc = jnp.maximum(acc, jnp.where(a != dist_old[ds], 1, 0))
            dist_old[ds] = a
            return acc
        accv = lax.fori_loop(0, _VPR, diff_j, jnp.zeros((_L,), jnp.int32))
        return it + 1, jnp.max(accv) > 0

    lax.while_loop(bf_cond, bf_body, (0, True))
    pltpu.sync_copy(dist_old, dist_hbm.at[wid])


def _sc_bf(knn_i_packed, knn_wT):
    mesh = plsc.VectorSubcoreMesh(core_axis_name="c", subcore_axis_name="s")
    f = pl.kernel(
        _bf_body,
        out_type=jax.ShapeDtypeStruct((N_SOURCES, _N), jnp.float32),
        mesh=mesh,
        scratch_types=[
            pltpu.VMEM((N_NEIGHBORS, _N // 2), jnp.int32),
            pltpu.VMEM((N_NEIGHBORS, _N), jnp.float32),
            pltpu.VMEM((_N,), jnp.float32),
            pltpu.VMEM((_N,), jnp.float32),
        ],
        compiler_params=pltpu.CompilerParams(needs_layout_passes=False,
                                             use_tc_tiling_on_sc=False),
    )
    return f(knn_i_packed, knn_wT)


def _sqrt_kernel(d2_blk, out):
    out[...] = jnp.sqrt(jnp.maximum(d2_blk[...], 0.0))


def _knn_sqrt(knn_d2):
    return pl.pallas_call(
        _sqrt_kernel,
        out_shape=jax.ShapeDtypeStruct(knn_d2.shape, jnp.float32),
    )(knn_d2)


def _loss_kernel(dist_blk, euc2_blk, out):
    dist = dist_blk[...]
    euc = jnp.sqrt(jnp.maximum(euc2_blk[...], 0.0))
    mask = (dist < INF * 0.5) & (euc > 1e-8)
    ratios = dist / jnp.maximum(euc, 1e-8)
    sq_err = jnp.where(mask, (ratios - TARGET_RATIO) ** 2, 0.0)
    cnt = jnp.maximum(jnp.sum(mask.astype(jnp.int32)), 1)
    loss = jnp.sum(sq_err) / cnt.astype(jnp.float32) * LAMBDA_REG
    out[...] = loss[None, None]


def _loss(dist, d2):
    return pl.pallas_call(
        _loss_kernel,
        grid=(1,),
        in_specs=[
            pl.BlockSpec((N_SOURCES, _N), lambda i: (0, 0)),
            pl.BlockSpec((N_SOURCES, _N), lambda i: (0, 0)),
        ],
        out_specs=pl.BlockSpec((1, 1), lambda i: (0, 0)),
        out_shape=jax.ShapeDtypeStruct((1, 1), jnp.float32),
    )(dist, d2)


def kernel(embeddings):
    x = lax.stop_gradient(embeddings)
    D2 = _pairwise_d2(x)
    knn_d2, knn_idx = _sc_topk(D2)
    knn_d = _knn_sqrt(knn_d2[:, 1:])
    knn_i = knn_idx[:, 1:]

    iT3 = knn_i.T.astype(jnp.int32).reshape(N_NEIGHBORS, _N // 32, 2, _L)
    ipacked = (iT3[:, :, 0, :] | (iT3[:, :, 1, :] << 16)).reshape(
        N_NEIGHBORS, _N // 2)
    dist = _sc_bf(ipacked, knn_d.T)
    loss = _loss(dist, D2[:N_SOURCES, :])
    return loss[0, 0]


# speedup vs baseline: 3.2331x; 1.0908x over previous
import functools

import jax
import jax.numpy as jnp
from jax import lax
from jax.experimental import pallas as pl
from jax.experimental.pallas import tpu as pltpu
from jax.experimental.pallas import tpu_sc as plsc

N_NEIGHBORS = 15
TARGET_RATIO = 1.8
LAMBDA_REG = 0.1
N_SOURCES = 32
N_BF_ITERS = 20
INF = 1e10

_N = 4096
_K = 128
_BR = 256

_TOPK = 16
_NW = 32
_ROWS_PER_W = _N // _NW
_CHUNK = 8
_N_CHUNKS = _ROWS_PER_W // _CHUNK
_L = 16
_VPR = _N // _L


def _d2_kernel(x_blk, xt_full, sq_blk, sq_full, out):
    acc = jnp.dot(x_blk[...], xt_full[...], preferred_element_type=jnp.float32)
    out[...] = sq_blk[...].T + sq_full[...] - 2.0 * acc


def _pairwise_d2(x):
    sq = jnp.sum(x * x, axis=1)
    return pl.pallas_call(
        _d2_kernel,
        grid=(_N // _BR,),
        in_specs=[
            pl.BlockSpec((_BR, _K), lambda i: (i, 0)),
            pl.BlockSpec((_K, _N), lambda i: (0, 0)),
            pl.BlockSpec((1, _BR), lambda i: (0, i)),
            pl.BlockSpec((1, _N), lambda i: (0, 0)),
        ],
        out_specs=pl.BlockSpec((_BR, _N), lambda i: (i, 0)),
        out_shape=jax.ShapeDtypeStruct((_N, _N), jnp.float32),
    )(x, x.T, sq[None, :], sq[None, :])


def _topk_body(d2_hbm, val_hbm, idx_hbm, buf, oval, oidx, tref, tiref, t15ref):
    wid = lax.axis_index("s") * 2 + lax.axis_index("c")
    lane = lax.iota(jnp.int32, _L)
    last_lane = lane == _L - 1

    def chunk_body(c, _):
        row_base = wid * _ROWS_PER_W + c * _CHUNK
        pltpu.sync_copy(d2_hbm.at[pl.ds(row_base, _CHUNK)], buf)

        def row_body(r, _):
            def one_merge(tval, tidx, v, i):
                vs, is_ = plsc.sort_key_val(v, i)
                rv = lax.rev(tval, (0,))
                ri = lax.rev(tidx, (0,))
                sel = vs <= rv
                lo = jnp.minimum(vs, rv)
                li = jnp.where(sel, is_, ri)
                return tuple(plsc.sort_key_val(lo, li))

            nch = 8

            def merge4(j, carry):
                out = []
                for q in range(nch):
                    tv, ti = carry[2 * q], carry[2 * q + 1]
                    base = (j * nch + q) * _L
                    v = buf[r, pl.ds(base, _L)]
                    tv, ti = one_merge(tv, ti, v, base + lane)
                    out += [tv, ti]
                return tuple(out)

            big = jnp.full((_L,), 1e30, jnp.float32)
            zero = jnp.zeros((_L,), jnp.int32)
            carry = lax.fori_loop(0, _VPR // nch, merge4,
                                  (big, zero) * nch)
            while len(carry) > 2:
                nxt = []
                for q in range(0, len(carry), 4):
                    tvA, tiA, tvB, tiB = carry[q:q + 4]
                    rv = lax.rev(tvB, (0,))
                    ri = lax.rev(tiB, (0,))
                    sel = tvA <= rv
                    lo = jnp.minimum(tvA, rv)
                    li = jnp.where(sel, tiA, ri)
                    nxt += list(plsc.sort_key_val(lo, li))
                carry = nxt
            oval[r, :] = carry[0]
            oidx[r, :] = carry[1]
            return 0

        lax.fori_loop(0, _CHUNK, row_body, 0)
        pltpu.sync_copy(oval, val_hbm.at[pl.ds(row_base, _CHUNK)])
        pltpu.sync_copy(oidx, idx_hbm.at[pl.ds(row_base, _CHUNK)])
        return 0

    lax.fori_loop(0, _N_CHUNKS, chunk_body, 0)


def _sc_topk(d2):
    mesh = plsc.VectorSubcoreMesh(core_axis_name="c", subcore_axis_name="s")
    f = pl.kernel(
        _topk_body,
        out_type=(
            jax.ShapeDtypeStruct((_N, _TOPK), jnp.float32),
            jax.ShapeDtypeStruct((_N, _TOPK), jnp.int32),
        ),
        mesh=mesh,
        scratch_types=[
            pltpu.VMEM((_CHUNK, _N), jnp.float32),
            pltpu.VMEM((_CHUNK, _TOPK), jnp.float32),
            pltpu.VMEM((_CHUNK, _TOPK), jnp.int32),
            pltpu.VMEM((_L,), jnp.float32),
            pltpu.VMEM((_L,), jnp.int32),
            pltpu.VMEM((_L,), jnp.float32),
        ],
        compiler_params=pltpu.CompilerParams(needs_layout_passes=False),
    )
    return f(d2)


_BIG = 1e30


def _bf_body(ip_hbm, wT_hbm, dist_hbm, idx_res, w_res, dist_old, dist_new):
    wid = lax.axis_index("s") * 2 + lax.axis_index("c")
    lane = lax.iota(jnp.int32, _L)

    pltpu.sync_copy(ip_hbm, idx_res)
    pltpu.sync_copy(wT_hbm, w_res)

    def init_j(j, _):
        dist_old[pl.ds(j * _L, _L)] = jnp.full((_L,), INF, jnp.float32)
        return 0
    lax.fori_loop(0, _VPR, init_j, 0)
    dist_old[pl.ds((wid // _L) * _L, _L)] = jnp.where(
        lane == wid % _L, 0.0, INF)

    def bf_cond(c):
        it, changed = c
        return (it < N_BF_ITERS) & changed

    def bf_body(c):
        it, _ = c

        def copy_j(j, _):
            ds = pl.ds(j * _L, _L)
            dist_new[ds] = dist_old[ds]
            return 0
        lax.fori_loop(0, _VPR, copy_j, 0)

        def j_body(j, _):
            dsA = pl.ds(j * 2 * _L, _L)
            dsB = pl.ds(j * 2 * _L + _L, _L)
            mnA = dist_new[dsA]
            mnB = dist_new[dsB]
            doA = dist_old[dsA]
            doB = dist_old[dsB]
            for k in range(N_NEIGHBORS):
                v32 = idx_res[k, pl.ds(j * _L, _L)]
                ia = v32 & 0xFFFF
                ib = lax.shift_right_logical(v32, 16)
                wA = w_res[k, dsA]
                wB = w_res[k, dsB]
                mnA = jnp.minimum(mnA, plsc.load_gather(dist_old, [ia]) + wA)
                mnB = jnp.minimum(mnB, plsc.load_gather(dist_old, [ib]) + wB)
                candA = doA + wA
                candB = doB + wB
                lostA = candA < plsc.load_gather(dist_new, [ia])
                lostB = candB < plsc.load_gather(dist_new, [ib])

                @pl.when(jnp.any(lostA | lostB))
                def _():
                    def wbody(cw):
                        la, lb = cw
                        plsc.store_scatter(dist_new, [ia], candA, mask=la)
                        plsc.store_scatter(dist_new, [ib], candB, mask=lb)
                        ra = plsc.load_gather(dist_new, [ia])
                        rb = plsc.load_gather(dist_new, [ib])
                        return candA < ra, candB < rb
                    lax.while_loop(lambda cw: jnp.any(cw[0] | cw[1]),
                                   wbody, (lostA, lostB))
            dist_new[dsA] = jnp.minimum(dist_new[dsA], mnA)
            dist_new[dsB] = jnp.minimum(dist_new[dsB], mnB)
            return 0
        lax.fori_loop(0, _VPR // 2, j_body, 0)

        def diff_j(j, acc):
            ds = pl.ds(j * _L, _L)
            a = dist_new[ds]
            acc = jnp.maximum(acc, jnp.where(a != dist_old[ds], 1, 0))
            dist_old[ds] = a
            return acc
        accv = lax.fori_loop(0, _VPR, diff_j, jnp.zeros((_L,), jnp.int32))
        return it + 1, jnp.max(accv) > 0

    lax.while_loop(bf_cond, bf_body, (0, True))
    pltpu.sync_copy(dist_old, dist_hbm.at[wid])


def _sc_bf(knn_i_packed, knn_wT):
    mesh = plsc.VectorSubcoreMesh(core_axis_name="c", subcore_axis_name="s")
    f = pl.kernel(
        _bf_body,
        out_type=jax.ShapeDtypeStruct((N_SOURCES, _N), jnp.float32),
        mesh=mesh,
        scratch_types=[
            pltpu.VMEM((N_NEIGHBORS, _N // 2), jnp.int32),
            pltpu.VMEM((N_NEIGHBORS, _N), jnp.float32),
            pltpu.VMEM((_N,), jnp.float32),
            pltpu.VMEM((_N,), jnp.float32),
        ],
        compiler_params=pltpu.CompilerParams(needs_layout_passes=False,
                                             use_tc_tiling_on_sc=False),
    )
    return f(knn_i_packed, knn_wT)


def _sqrt_kernel(d2_blk, out):
    out[...] = jnp.sqrt(jnp.maximum(d2_blk[...], 0.0))


def _knn_sqrt(knn_d2):
    return pl.pallas_call(
        _sqrt_kernel,
        out_shape=jax.ShapeDtypeStruct(knn_d2.shape, jnp.float32),
    )(knn_d2)


def _loss_kernel(dist_blk, euc2_blk, out):
    dist = dist_blk[...]
    euc = jnp.sqrt(jnp.maximum(euc2_blk[...], 0.0))
    mask = (dist < INF * 0.5) & (euc > 1e-8)
    ratios = dist / jnp.maximum(euc, 1e-8)
    sq_err = jnp.where(mask, (ratios - TARGET_RATIO) ** 2, 0.0)
    cnt = jnp.maximum(jnp.sum(mask.astype(jnp.int32)), 1)
    loss = jnp.sum(sq_err) / cnt.astype(jnp.float32) * LAMBDA_REG
    out[...] = loss[None, None]


def _loss(dist, d2):
    return pl.pallas_call(
        _loss_kernel,
        grid=(1,),
        in_specs=[
            pl.BlockSpec((N_SOURCES, _N), lambda i: (0, 0)),
            pl.BlockSpec((N_SOURCES, _N), lambda i: (0, 0)),
        ],
        out_specs=pl.BlockSpec((1, 1), lambda i: (0, 0)),
        out_shape=jax.ShapeDtypeStruct((1, 1), jnp.float32),
    )(dist, d2)


def kernel(embeddings):
    x = lax.stop_gradient(embeddings)
    D2 = _pairwise_d2(x)
    knn_d2, knn_idx = _sc_topk(D2)
    knn_d = _knn_sqrt(knn_d2[:, 1:])
    knn_i = knn_idx[:, 1:]

    iT3 = knn_i.T.astype(jnp.int32).reshape(N_NEIGHBORS, _N // 32, 2, _L)
    ipacked = (iT3[:, :, 0, :] | (iT3[:, :, 1, :] << 16)).reshape(
        N_NEIGHBORS, _N // 2)
    dist = _sc_bf(ipacked, knn_d.T)
    loss = _loss(dist, D2[:N_SOURCES, :])
    return loss[0, 0]


# speedup vs baseline: 4.0054x; 1.2389x over previous
import functools

import jax
import jax.numpy as jnp
from jax import lax
from jax.experimental import pallas as pl
from jax.experimental.pallas import tpu as pltpu
from jax.experimental.pallas import tpu_sc as plsc

N_NEIGHBORS = 15
TARGET_RATIO = 1.8
LAMBDA_REG = 0.1
N_SOURCES = 32
N_BF_ITERS = 20
INF = 1e10

_N = 4096
_K = 128
_BR = 256

_TOPK = 16
_NW = 32
_ROWS_PER_W = _N // _NW
_CHUNK = 8
_N_CHUNKS = _ROWS_PER_W // _CHUNK
_L = 16
_VPR = _N // _L


def _d2_kernel(x_blk, xt_full, sq_blk, sq_full, out):
    acc = jnp.dot(x_blk[...], xt_full[...], preferred_element_type=jnp.float32)
    out[...] = sq_blk[...].T + sq_full[...] - 2.0 * acc


def _pairwise_d2(x):
    sq = jnp.sum(x * x, axis=1)
    return pl.pallas_call(
        _d2_kernel,
        grid=(_N // _BR,),
        in_specs=[
            pl.BlockSpec((_BR, _K), lambda i: (i, 0)),
            pl.BlockSpec((_K, _N), lambda i: (0, 0)),
            pl.BlockSpec((1, _BR), lambda i: (0, i)),
            pl.BlockSpec((1, _N), lambda i: (0, 0)),
        ],
        out_specs=pl.BlockSpec((_BR, _N), lambda i: (i, 0)),
        out_shape=jax.ShapeDtypeStruct((_N, _N), jnp.float32),
    )(x, x.T, sq[None, :], sq[None, :])


def _topk_body(d2_hbm, val_hbm, idx_hbm, buf, oval, oidx, tref, tiref, t15ref):
    wid = lax.axis_index("s") * 2 + lax.axis_index("c")
    lane = lax.iota(jnp.int32, _L)
    last_lane = lane == _L - 1

    def chunk_body(c, _):
        row_base = wid * _ROWS_PER_W + c * _CHUNK
        pltpu.sync_copy(d2_hbm.at[pl.ds(row_base, _CHUNK)], buf)

        def row_body(r, _):
            def one_merge(tval, tidx, v, i):
                vs, is_ = plsc.sort_key_val(v, i)
                rv = lax.rev(tval, (0,))
                ri = lax.rev(tidx, (0,))
                sel = vs <= rv
                lo = jnp.minimum(vs, rv)
                li = jnp.where(sel, is_, ri)
                return tuple(plsc.sort_key_val(lo, li))

            nch = 8

            def merge4(j, carry):
                out = []
                for q in range(nch):
                    tv, ti = carry[2 * q], carry[2 * q + 1]
                    base = (j * nch + q) * _L
                    v = buf[r, pl.ds(base, _L)]
                    tv, ti = one_merge(tv, ti, v, base + lane)
                    out += [tv, ti]
                return tuple(out)

            big = jnp.full((_L,), 1e30, jnp.float32)
            zero = jnp.zeros((_L,), jnp.int32)
            carry = lax.fori_loop(0, _VPR // nch, merge4,
                                  (big, zero) * nch)
            while len(carry) > 2:
                nxt = []
                for q in range(0, len(carry), 4):
                    tvA, tiA, tvB, tiB = carry[q:q + 4]
                    rv = lax.rev(tvB, (0,))
                    ri = lax.rev(tiB, (0,))
                    sel = tvA <= rv
                    lo = jnp.minimum(tvA, rv)
                    li = jnp.where(sel, tiA, ri)
                    nxt += list(plsc.sort_key_val(lo, li))
                carry = nxt
            oval[r, :] = carry[0]
            oidx[r, :] = carry[1]
            return 0

        lax.fori_loop(0, _CHUNK, row_body, 0)
        pltpu.sync_copy(oval, val_hbm.at[pl.ds(row_base, _CHUNK)])
        pltpu.sync_copy(oidx, idx_hbm.at[pl.ds(row_base, _CHUNK)])
        return 0

    lax.fori_loop(0, _N_CHUNKS, chunk_body, 0)


def _sc_topk(d2):
    mesh = plsc.VectorSubcoreMesh(core_axis_name="c", subcore_axis_name="s")
    f = pl.kernel(
        _topk_body,
        out_type=(
            jax.ShapeDtypeStruct((_N, _TOPK), jnp.float32),
            jax.ShapeDtypeStruct((_N, _TOPK), jnp.int32),
        ),
        mesh=mesh,
        scratch_types=[
            pltpu.VMEM((_CHUNK, _N), jnp.float32),
            pltpu.VMEM((_CHUNK, _TOPK), jnp.float32),
            pltpu.VMEM((_CHUNK, _TOPK), jnp.int32),
            pltpu.VMEM((_L,), jnp.float32),
            pltpu.VMEM((_L,), jnp.int32),
            pltpu.VMEM((_L,), jnp.float32),
        ],
        compiler_params=pltpu.CompilerParams(needs_layout_passes=False),
    )
    return f(d2)


_BIG = 1e30


def _bf_body(ip_hbm, wT_hbm, dist_hbm, idx_res, w_res, dist_old, dist_new):
    wid = lax.axis_index("s") * 2 + lax.axis_index("c")
    lane = lax.iota(jnp.int32, _L)

    pltpu.sync_copy(ip_hbm, idx_res)
    pltpu.sync_copy(wT_hbm, w_res)

    def init_j(j, _):
        dist_old[pl.ds(j * _L, _L)] = jnp.full((_L,), INF, jnp.float32)
        return 0
    lax.fori_loop(0, _VPR, init_j, 0)
    dist_old[pl.ds((wid // _L) * _L, _L)] = jnp.where(
        lane == wid % _L, 0.0, INF)

    def copy_j(j, _):
        ds = pl.ds(j * _L, _L)
        dist_new[ds] = dist_old[ds]
        return 0
    lax.fori_loop(0, _VPR, copy_j, 0)

    def bf_cond(c):
        it, changed = c
        return (it < N_BF_ITERS) & (changed > 0)

    def bf_body(c):
        it, _ = c

        def j_body(j, pend):
            dsA = pl.ds(j * 2 * _L, _L)
            dsB = pl.ds(j * 2 * _L + _L, _L)
            mnA = dist_new[dsA]
            mnB = dist_new[dsB]
            doA = dist_old[dsA]
            doB = dist_old[dsB]
            for k in range(N_NEIGHBORS):
                v32 = idx_res[k, pl.ds(j * _L, _L)]
                ia = v32 & 0xFFFF
                ib = lax.shift_right_logical(v32, 16)
                wA = w_res[k, dsA]
                wB = w_res[k, dsB]
                mnA = jnp.minimum(mnA, plsc.load_gather(dist_old, [ia]) + wA)
                mnB = jnp.minimum(mnB, plsc.load_gather(dist_old, [ib]) + wB)
                candA = doA + wA
                candB = doB + wB
                lostA = candA < plsc.load_gather(dist_new, [ia])
                lostB = candB < plsc.load_gather(dist_new, [ib])
                plsc.store_scatter(dist_new, [ia], candA, mask=lostA)
                plsc.store_scatter(dist_new, [ib], candB, mask=lostB)
                stillA = candA < plsc.load_gather(dist_new, [ia])
                stillB = candB < plsc.load_gather(dist_new, [ib])
                pend = pend | stillA | stillB
            dist_new[dsA] = jnp.minimum(dist_new[dsA], mnA)
            dist_new[dsB] = jnp.minimum(dist_new[dsB], mnB)
            return pend
        pend = lax.fori_loop(0, _VPR // 2, j_body,
                             jnp.zeros((_L,), jnp.bool_))

        def commit(_):
            def diff_j(j, acc):
                ds = pl.ds(j * _L, _L)
                a = dist_new[ds]
                acc = jnp.maximum(acc, jnp.where(a != dist_old[ds], 1, 0))
                dist_old[ds] = a
                return acc
            accv = lax.fori_loop(0, _VPR, diff_j, jnp.zeros((_L,), jnp.int32))
            return it + 1, jnp.max(accv)

        def redo(_):
            return it, jnp.int32(1)

        return lax.cond(jnp.any(pend), redo, commit, 0)

    lax.while_loop(bf_cond, bf_body, (0, jnp.int32(1)))
    pltpu.sync_copy(dist_old, dist_hbm.at[wid])


def _sc_bf(knn_i_packed, knn_wT):
    mesh = plsc.VectorSubcoreMesh(core_axis_name="c", subcore_axis_name="s")
    f = pl.kernel(
        _bf_body,
        out_type=jax.ShapeDtypeStruct((N_SOURCES, _N), jnp.float32),
        mesh=mesh,
        scratch_types=[
            pltpu.VMEM((N_NEIGHBORS, _N // 2), jnp.int32),
            pltpu.VMEM((N_NEIGHBORS, _N), jnp.float32),
            pltpu.VMEM((_N,), jnp.float32),
            pltpu.VMEM((_N,), jnp.float32),
        ],
        compiler_params=pltpu.CompilerParams(needs_layout_passes=False,
                                             use_tc_tiling_on_sc=False),
    )
    return f(knn_i_packed, knn_wT)


def _sqrt_kernel(d2_blk, out):
    out[...] = jnp.sqrt(jnp.maximum(d2_blk[...], 0.0))


def _knn_sqrt(knn_d2):
    return pl.pallas_call(
        _sqrt_kernel,
        out_shape=jax.ShapeDtypeStruct(knn_d2.shape, jnp.float32),
    )(knn_d2)


def _loss_kernel(dist_blk, euc2_blk, out):
    dist = dist_blk[...]
    euc = jnp.sqrt(jnp.maximum(euc2_blk[...], 0.0))
    mask = (dist < INF * 0.5) & (euc > 1e-8)
    ratios = dist / jnp.maximum(euc, 1e-8)
    sq_err = jnp.where(mask, (ratios - TARGET_RATIO) ** 2, 0.0)
    cnt = jnp.maximum(jnp.sum(mask.astype(jnp.int32)), 1)
    loss = jnp.sum(sq_err) / cnt.astype(jnp.float32) * LAMBDA_REG
    out[...] = loss[None, None]


def _loss(dist, d2):
    return pl.pallas_call(
        _loss_kernel,
        grid=(1,),
        in_specs=[
            pl.BlockSpec((N_SOURCES, _N), lambda i: (0, 0)),
            pl.BlockSpec((N_SOURCES, _N), lambda i: (0, 0)),
        ],
        out_specs=pl.BlockSpec((1, 1), lambda i: (0, 0)),
        out_shape=jax.ShapeDtypeStruct((1, 1), jnp.float32),
    )(dist, d2)


def kernel(embeddings):
    x = lax.stop_gradient(embeddings)
    D2 = _pairwise_d2(x)
    knn_d2, knn_idx = _sc_topk(D2)
    knn_d = _knn_sqrt(knn_d2[:, 1:])
    knn_i = knn_idx[:, 1:]

    iT3 = knn_i.T.astype(jnp.int32).reshape(N_NEIGHBORS, _N // 32, 2, _L)
    ipacked = (iT3[:, :, 0, :] | (iT3[:, :, 1, :] << 16)).reshape(
        N_NEIGHBORS, _N // 2)
    dist = _sc_bf(ipacked, knn_d.T)
    loss = _loss(dist, D2[:N_SOURCES, :])
    return loss[0, 0]
